# baseline (speedup 1.0000x reference)
import functools

import numpy as np
import jax
import jax.numpy as jnp
from jax import lax
from jax.experimental import pallas as pl
from jax.experimental.pallas import tpu as pltpu

N = 50000
M = 10000
MP = 10240
K = 32
RADIUS = 0.06
R2 = RADIUS * RADIUS
NUM_FREQ = 8
EMBED_DIM = 48
BM = 128


def _embed_consts():
    freqs = 1.0 / (10000.0 ** (np.arange(NUM_FREQ, dtype=np.float64) / NUM_FREQ))
    sel16 = np.zeros((16, EMBED_DIM), dtype=np.float32)
    selx = np.zeros((3, EMBED_DIM), dtype=np.float32)
    for c in range(3):
        for t in range(16):
            sel16[c, 16 * c + t] = freqs[t % 8]
            selx[c, 16 * c + t] = freqs[t % 8]
    return jnp.asarray(sel16), jnp.asarray(selx)


def _emb_from_ang(ang):
    j = lax.broadcasted_iota(jnp.int32, ang.shape, len(ang.shape) - 1)
    return jnp.where((j % 16) < 8, jnp.cos(ang), jnp.sin(ang))


def _mlp_body(gy_ref, gf_ref, x_ref, m_ref, sel16_ref, selx_ref,
              w0_ref, b0_ref, w1_ref, b1_ref, w2_ref, b2_ref, o_ref):
    f32 = jnp.float32
    gy = gy_ref[...]
    ang_y = jnp.dot(gy, sel16_ref[...], preferred_element_type=f32)
    emb_y = _emb_from_ang(ang_y)

    xb = x_ref[...]
    ang_x = jnp.dot(xb, selx_ref[...], preferred_element_type=f32)
    emb_x = _emb_from_ang(ang_x)

    w0 = w0_ref[...]
    w0y = w0[:EMBED_DIM, :]
    w0x = w0[EMBED_DIM:, :]
    t_x = jnp.dot(emb_x, w0x, preferred_element_type=f32)
    t_x = jnp.broadcast_to(t_x[:, None, :], (BM, K, 64)).reshape(BM * K, 64)

    h = jnp.dot(emb_y, w0y, preferred_element_type=f32) + t_x + b0_ref[...]
    h = jax.nn.gelu(h)
    h = jnp.dot(h, w1_ref[...], preferred_element_type=f32) + b1_ref[...]
    h = jax.nn.gelu(h)
    kv = jnp.dot(h, w2_ref[...], preferred_element_type=f32) + b2_ref[...]

    contrib = kv * gf_ref[...]
    msk = m_ref[...].reshape(BM, K, 1)
    o_ref[...] = jnp.sum(contrib.reshape(BM, K, 128) * msk, axis=1)


def _mlp_call(g_y, g_f, xp, msk, W0, b0, W1, b1, W2, b2, sel16, selx):
    nb = MP // BM
    full = lambda shape: pl.BlockSpec(shape, lambda i: tuple(0 for _ in shape))
    return pl.pallas_call(
        _mlp_body,
        grid=(nb,),
        in_specs=[
            pl.BlockSpec((BM * K, 16), lambda i: (i, 0)),
            pl.BlockSpec((BM * K, 128), lambda i: (i, 0)),
            pl.BlockSpec((BM, 3), lambda i: (i, 0)),
            pl.BlockSpec((BM, K), lambda i: (i, 0)),
            full((16, EMBED_DIM)),
            full((3, EMBED_DIM)),
            full((2 * EMBED_DIM, 64)),
            full((64,)),
            full((64, 64)),
            full((64,)),
            full((64, 128)),
            full((128,)),
        ],
        out_specs=pl.BlockSpec((BM, 128), lambda i: (i, 0)),
        out_shape=jax.ShapeDtypeStruct((MP, 128), jnp.float32),
    )(g_y, g_f, xp, msk, sel16, selx, W0, b0, W1, b1, W2, b2)


def _nbr_search_jnp(y, x):
    data_sq = jnp.sum(y * y, axis=1)
    idx_chunks, mask_chunks = [], []
    for s in range(0, x.shape[0], 2048):
        q = x[s:s + 2048]
        d2 = jnp.sum(q * q, axis=1)[:, None] + data_sq[None, :] - 2.0 * (q @ y.T)
        neg_d, idx = jax.lax.top_k(-d2, K)
        idx_chunks.append(idx)
        mask_chunks.append((-neg_d) <= R2)
    return jnp.concatenate(idx_chunks, axis=0), jnp.concatenate(mask_chunks, axis=0)


def kernel(y, x, f_y, W0, b0, W1, b1, W2, b2):
    sel16, selx = _embed_consts()
    y_pad16 = jnp.pad(y, ((0, 0), (0, 13)))

    nbr_idx, nbr_mask = _nbr_search_jnp(y, x)

    nbr_idx = jnp.pad(nbr_idx, ((0, MP - M), (0, 0)))
    msk = jnp.pad(nbr_mask.astype(jnp.float32), ((0, MP - M), (0, 0)))
    xp = jnp.pad(x, ((0, MP - M), (0, 0)), constant_values=2.0)

    flat_idx = nbr_idx.reshape(MP * K)
    g_y = jnp.take(y_pad16, flat_idx, axis=0)
    g_f = jnp.take(f_y, flat_idx, axis=0)

    out = _mlp_call(g_y, g_f, xp, msk, W0, b0, W1, b1, W2, b2, sel16, selx)
    return out[:M]


# speedup vs baseline: 1.0273x; 1.0273x over previous
import functools

import numpy as np
import jax
import jax.numpy as jnp
from jax import lax
from jax.experimental import pallas as pl
from jax.experimental.pallas import tpu as pltpu

N = 50000
M = 10000
MP = 10240
K = 32
RADIUS = 0.06
R2 = RADIUS * RADIUS
NUM_FREQ = 8
EMBED_DIM = 48
BM = 128


def _embed_consts():
    freqs = 1.0 / (10000.0 ** (np.arange(NUM_FREQ, dtype=np.float64) / NUM_FREQ))
    sel16 = np.zeros((16, EMBED_DIM), dtype=np.float32)
    selx = np.zeros((3, EMBED_DIM), dtype=np.float32)
    for c in range(3):
        for t in range(16):
            sel16[c, 16 * c + t] = freqs[t % 8]
            selx[c, 16 * c + t] = freqs[t % 8]
    return jnp.asarray(sel16), jnp.asarray(selx)


def _emb_from_ang(ang):
    j = lax.broadcasted_iota(jnp.int32, ang.shape, len(ang.shape) - 1)
    return jnp.where((j % 16) < 8, jnp.cos(ang), jnp.sin(ang))


def _mlp_body(sx_ref, sy_ref, sz_ref, gf_ref, x_ref, m_ref, selx_ref,
              w0_ref, b0_ref, w1_ref, b1_ref, w2_ref, b2_ref, o_ref):
    f32 = jnp.float32
    fr = selx_ref[...]
    ang_y = (sx_ref[...][:, :, None] * fr[0:1, :][None, :, :]
             + sy_ref[...][:, :, None] * fr[1:2, :][None, :, :]
             + sz_ref[...][:, :, None] * fr[2:3, :][None, :, :])
    emb_y = _emb_from_ang(ang_y).reshape(BM * K, EMBED_DIM)

    xb = x_ref[...]
    ang_x = jnp.dot(xb, selx_ref[...], preferred_element_type=f32)
    emb_x = _emb_from_ang(ang_x)

    w0 = w0_ref[...]
    w0y = w0[:EMBED_DIM, :]
    w0x = w0[EMBED_DIM:, :]
    t_x = jnp.dot(emb_x, w0x, preferred_element_type=f32)
    t_x = jnp.broadcast_to(t_x[:, None, :], (BM, K, 64)).reshape(BM * K, 64)

    h = jnp.dot(emb_y, w0y, preferred_element_type=f32) + t_x + b0_ref[...]
    h = jax.nn.gelu(h)
    h = jnp.dot(h, w1_ref[...], preferred_element_type=f32) + b1_ref[...]
    h = jax.nn.gelu(h)
    kv = jnp.dot(h, w2_ref[...], preferred_element_type=f32) + b2_ref[...]

    contrib = kv * gf_ref[...]
    msk = m_ref[...].reshape(BM, K, 1)
    o_ref[...] = jnp.sum(contrib.reshape(BM, K, 128) * msk, axis=1)


def _mlp_call(s_x, s_y, s_z, g_f, xp, msk, W0, b0, W1, b1, W2, b2, selx):
    nb = MP // BM
    full = lambda shape: pl.BlockSpec(shape, lambda i: tuple(0 for _ in shape))
    return pl.pallas_call(
        _mlp_body,
        grid=(nb,),
        in_specs=[
            pl.BlockSpec((BM, K), lambda i: (i, 0)),
            pl.BlockSpec((BM, K), lambda i: (i, 0)),
            pl.BlockSpec((BM, K), lambda i: (i, 0)),
            pl.BlockSpec((BM * K, 128), lambda i: (i, 0)),
            pl.BlockSpec((BM, 3), lambda i: (i, 0)),
            pl.BlockSpec((BM, K), lambda i: (i, 0)),
            full((3, EMBED_DIM)),
            full((2 * EMBED_DIM, 64)),
            full((64,)),
            full((64, 64)),
            full((64,)),
            full((64, 128)),
            full((128,)),
        ],
        out_specs=pl.BlockSpec((BM, 128), lambda i: (i, 0)),
        out_shape=jax.ShapeDtypeStruct((MP, 128), jnp.float32),
    )(s_x, s_y, s_z, g_f, xp, msk, selx, W0, b0, W1, b1, W2, b2)


def _gather_call(f_y, flat_idx):
    from jax.experimental.pallas import tpu_sc as plsc

    total = MP * K
    nw = 32
    per_w = total // nw
    ck = 128
    nck = per_w // ck

    mesh = plsc.VectorSubcoreMesh(core_axis_name="c", subcore_axis_name="s")

    @functools.partial(
        pl.kernel, mesh=mesh,
        out_type=jax.ShapeDtypeStruct((total, 128), jnp.float32),
        scratch_types=[
            pltpu.VMEM((ck,), jnp.int32),
            pltpu.VMEM((ck, 128), jnp.float32),
            pltpu.SemaphoreType.DMA,
        ],
    )
    def k(f_hbm, idx_hbm, of_hbm, idx_v, rf_v, sem):
        wid = lax.axis_index("s") * 2 + lax.axis_index("c")
        base_w = wid * per_w

        def body(i, _):
            base = base_w + i * ck
            pltpu.sync_copy(idx_hbm.at[pl.ds(base, ck)], idx_v)
            pltpu.async_copy(f_hbm.at[idx_v], rf_v, sem).wait()
            pltpu.sync_copy(rf_v, of_hbm.at[pl.ds(base, ck)])
            return 0

        lax.fori_loop(0, nck, body, 0)

    return k(f_y, flat_idx)


def _nbr_search_jnp(y, x):
    data_sq = jnp.sum(y * y, axis=1)
    idx_chunks, mask_chunks = [], []
    for s in range(0, x.shape[0], 2048):
        q = x[s:s + 2048]
        d2 = jnp.sum(q * q, axis=1)[:, None] + data_sq[None, :] - 2.0 * (q @ y.T)
        neg_d, idx = jax.lax.top_k(-d2, K)
        idx_chunks.append(idx)
        mask_chunks.append((-neg_d) <= R2)
    return jnp.concatenate(idx_chunks, axis=0), jnp.concatenate(mask_chunks, axis=0)


def kernel(y, x, f_y, W0, b0, W1, b1, W2, b2):
    _, selx = _embed_consts()

    nbr_idx, nbr_mask = _nbr_search_jnp(y, x)

    nbr_idx = jnp.pad(nbr_idx, ((0, MP - M), (0, 0)))
    msk = jnp.pad(nbr_mask.astype(jnp.float32), ((0, MP - M), (0, 0)))
    xp = jnp.pad(x, ((0, MP - M), (0, 0)), constant_values=2.0)

    sel_c = jnp.take(y, nbr_idx.reshape(-1), axis=0).reshape(MP, K, 3)
    s_x, s_y, s_z = sel_c[..., 0], sel_c[..., 1], sel_c[..., 2]

    flat_idx = nbr_idx.reshape(MP * K)
    g_f = _gather_call(f_y, flat_idx)

    out = _mlp_call(s_x, s_y, s_z, g_f, xp, msk, W0, b0, W1, b1, W2, b2, selx)
    return out[:M]
